# baseline (speedup 1.0000x reference)
import jax
import jax.numpy as jnp
from jax.experimental import pallas as pl
from jax.experimental.pallas import tpu as pltpu

N = 2048
D = 256
K = 512
HB = N // 2


def _kmeans_kernel(x_hbm, y_ref, c_ref, ynew_ref, loss_ref,
                   xb0, xb1, sem0, sem1):
    cp0 = pltpu.make_async_copy(x_hbm.at[pl.ds(0, HB), :], xb0, sem0)
    cp1 = pltpu.make_async_copy(x_hbm.at[pl.ds(HB, HB), :], xb1, sem1)
    cp0.start()
    cp1.start()

    c = c_ref[...]
    c2x = c + c
    c2 = jnp.sum(c * c, axis=1, keepdims=True)
    yb = y_ref[...]
    row = jax.lax.broadcasted_iota(jnp.int32, (K, HB), 0)

    total = jnp.zeros((1, 1), jnp.float32)
    for h, (cp, xb) in enumerate(((cp0, xb0), (cp1, xb1))):
        cp.wait()
        x = xb[...]

        st = jax.lax.dot_general(
            c2x, x,
            dimension_numbers=(((1,), (1,)), ((), ())),
            preferred_element_type=jnp.float32,
            precision=jax.lax.Precision.HIGHEST,
        )
        s = c2 - st

        smin = jnp.min(s, axis=0, keepdims=True)
        ynew_ref[0:1, h * HB:(h + 1) * HB] = jnp.min(
            jnp.where(s == smin, row, K), axis=0, keepdims=True)

        hit = jnp.where(row == yb[0:1, h * HB:(h + 1) * HB], s, 0.0)
        total = total + (jnp.sum(x * x, axis=(0, 1), keepdims=True)
                         + jnp.sum(hit, axis=(0, 1), keepdims=True))

    loss_ref[...] = total


def kernel(x, y, centers):
    y2 = y.reshape(1, N)
    ynew2, loss2 = pl.pallas_call(
        _kmeans_kernel,
        in_specs=[
            pl.BlockSpec(memory_space=pltpu.MemorySpace.HBM),
            pl.BlockSpec(memory_space=pltpu.VMEM),
            pl.BlockSpec(memory_space=pltpu.VMEM),
        ],
        out_specs=(
            pl.BlockSpec(memory_space=pltpu.VMEM),
            pl.BlockSpec(memory_space=pltpu.VMEM),
        ),
        out_shape=(
            jax.ShapeDtypeStruct((1, N), jnp.int32),
            jax.ShapeDtypeStruct((1, 1), jnp.float32),
        ),
        scratch_shapes=[
            pltpu.VMEM((HB, D), jnp.float32),
            pltpu.VMEM((HB, D), jnp.float32),
            pltpu.SemaphoreType.DMA,
            pltpu.SemaphoreType.DMA,
        ],
    )(x, y2, centers)
    return (loss2[0, 0], ynew2.reshape(N))


# speedup vs baseline: 1.1284x; 1.1284x over previous
import jax
import jax.numpy as jnp
from jax.experimental import pallas as pl

N = 2048
D = 256
K = 512


def _kmeans_kernel(x_ref, y_ref, c_ref, ynew_ref, loss_ref):
    x = x_ref[...]
    c = c_ref[...]
    yb = y_ref[...]

    st = jax.lax.dot_general(
        c + c, x,
        dimension_numbers=(((1,), (1,)), ((), ())),
        preferred_element_type=jnp.float32,
        precision=jax.lax.Precision.HIGHEST,
    )
    c2 = jnp.sum(c * c, axis=1, keepdims=True)
    s = c2 - st

    smin = jnp.min(s, axis=0, keepdims=True)
    row = jax.lax.broadcasted_iota(jnp.int32, (K, N), 0)
    ynew_ref[...] = jnp.min(jnp.where(s == smin, row, K), axis=0, keepdims=True)

    hit = jnp.where(row == yb, s, 0.0)
    loss_ref[...] = (jnp.sum(x * x, axis=(0, 1), keepdims=True)
                     + jnp.sum(hit, axis=(0, 1), keepdims=True))


def kernel(x, y, centers):
    y2 = y.reshape(1, N)
    ynew2, loss2 = pl.pallas_call(
        _kmeans_kernel,
        out_shape=(
            jax.ShapeDtypeStruct((1, N), jnp.int32),
            jax.ShapeDtypeStruct((1, 1), jnp.float32),
        ),
    )(x, y2, centers)
    return (loss2[0, 0], ynew2.reshape(N))


# speedup vs baseline: 1.1428x; 1.0128x over previous
import jax
import jax.numpy as jnp
from jax.experimental import pallas as pl

N = 2048
D = 256
K = 512


def _kmeans_kernel(x_ref, y_ref, c_ref, ynew_ref, loss_ref):
    x = x_ref[...]
    c = c_ref[...]
    yb = y_ref[...]

    st = jax.lax.dot_general(
        c + c, x,
        dimension_numbers=(((1,), (1,)), ((), ())),
        preferred_element_type=jnp.float32,
        precision=jax.lax.Precision.HIGHEST,
    )
    c2 = jnp.sum(c * c, axis=1, keepdims=True)
    s = c2 - st

    ynew_ref[...] = jnp.argmin(s, axis=0, keepdims=True).astype(jnp.int32)

    row = jax.lax.broadcasted_iota(jnp.int32, (K, N), 0)
    hit = jnp.where(row == yb, s, 0.0)
    loss_ref[...] = (jnp.sum(x * x, axis=(0, 1), keepdims=True)
                     + jnp.sum(hit, axis=(0, 1), keepdims=True))


def kernel(x, y, centers):
    y2 = y.reshape(1, N)
    ynew2, loss2 = pl.pallas_call(
        _kmeans_kernel,
        out_shape=(
            jax.ShapeDtypeStruct((1, N), jnp.int32),
            jax.ShapeDtypeStruct((1, 1), jnp.float32),
        ),
    )(x, y2, centers)
    return (loss2[0, 0], ynew2.reshape(N))
